# Initial kernel scaffold; baseline (speedup 1.0000x reference)
import jax
import jax.numpy as jnp
from jax.experimental import pallas as pl


def kernel(embeddings, target, triplets):
    raise NotImplementedError("write your pallas kernel here")


# speedup vs baseline: 13.3649x; 13.3649x over previous
import functools

import jax
import jax.numpy as jnp
from jax import lax
from jax.experimental import pallas as pl
from jax.experimental.pallas import tpu as pltpu
from jax.experimental.pallas import tpu_sc as plsc

_MARGIN = 0.2
_BETA = 1.2

_NC = 2
_NS = 16
_NW = _NC * _NS
_L = 16

_N = 16384
_D = 16
_T = 65536
_TW = _T // _NW
_CH = 128
_NCHUNK = _TW // _CH


def _sqrt16(x):
    i = plsc.bitcast(x, jnp.int32)
    i = jnp.int32(0x5F3759DF) - (i >> 1)
    y = plsc.bitcast(i, jnp.float32)
    xh = x * 0.5
    y = y * (1.5 - xh * y * y)
    y = y * (1.5 - xh * y * y)
    y = y * (1.5 - xh * y * y)
    return x * y


def _margin_body(emb, anc, pos, neg, out, idx_a, idx_p, idx_n,
                 rows_a, rows_p, rows_n, obuf, sem):
    wid = lax.axis_index("s") * _NC + lax.axis_index("c")
    base = wid * _TW

    pltpu.sync_copy(anc.at[pl.ds(base, _TW)], idx_a)
    pltpu.sync_copy(pos.at[pl.ds(base, _TW)], idx_p)
    pltpu.sync_copy(neg.at[pl.ds(base, _TW)], idx_n)

    def fire(c, carry):
        o = c * _CH
        pltpu.async_copy(emb.at[idx_a.at[pl.ds(o, _CH)]],
                         rows_a.at[pl.ds(o, _CH)], sem)
        pltpu.async_copy(emb.at[idx_p.at[pl.ds(o, _CH)]],
                         rows_p.at[pl.ds(o, _CH)], sem)
        pltpu.async_copy(emb.at[idx_n.at[pl.ds(o, _CH)]],
                         rows_n.at[pl.ds(o, _CH)], sem)
        return carry

    lax.fori_loop(0, _NCHUNK, fire, 0)

    def drain(c, carry):
        o = c * _CH
        pltpu.make_async_copy(emb.at[idx_a.at[pl.ds(o, _CH)]],
                              rows_a.at[pl.ds(o, _CH)], sem).wait()
        pltpu.make_async_copy(emb.at[idx_p.at[pl.ds(o, _CH)]],
                              rows_p.at[pl.ds(o, _CH)], sem).wait()
        pltpu.make_async_copy(emb.at[idx_n.at[pl.ds(o, _CH)]],
                              rows_n.at[pl.ds(o, _CH)], sem).wait()
        return carry

    lax.fori_loop(0, _NCHUNK, drain, 0)

    lanes = jnp.arange(_L, dtype=jnp.int32)
    zero = jnp.zeros((_L,), jnp.float32)

    def group(g, carry):
        asum, acnt = carry
        t0 = g * _L
        sap = zero
        san = zero
        for i in range(_L):
            va = rows_a[t0 + i, :]
            dap = va - rows_p[t0 + i, :]
            dan = va - rows_n[t0 + i, :]
            sap = jnp.where(lanes == i, jnp.sum(dap * dap), sap)
            san = jnp.where(lanes == i, jnp.sum(dan * dan), san)
        x_ap = sap + 1e-6
        x_an = san + 1e-6
        d_ap = _sqrt16(x_ap)
        d_an = _sqrt16(x_an)
        p_l = jnp.maximum(d_ap - (_BETA - _MARGIN), 0.0)
        n_l = jnp.maximum((_BETA + _MARGIN) - d_an, 0.0)
        hit = (p_l > 0.0) | (n_l > 0.0)
        asum = asum + (p_l + n_l)
        acnt = acnt + jnp.where(hit, 1.0, 0.0)
        return (asum, acnt)

    asum, acnt = lax.fori_loop(0, _TW // _L, group, (zero, zero))

    ssum = jnp.sum(asum)
    scnt = jnp.sum(acnt)
    obuf[...] = jnp.where(lanes == 0, ssum, jnp.where(lanes == 1, scnt, 0.0))
    pltpu.sync_copy(obuf, out.at[wid])


@functools.partial(
    pl.kernel,
    out_type=jax.ShapeDtypeStruct((_NW, _L), jnp.float32),
    mesh=plsc.VectorSubcoreMesh(core_axis_name="c", subcore_axis_name="s"),
    compiler_params=pltpu.CompilerParams(
        needs_layout_passes=False, use_tc_tiling_on_sc=False),
    scratch_types=[
        pltpu.VMEM((_TW,), jnp.int32),
        pltpu.VMEM((_TW,), jnp.int32),
        pltpu.VMEM((_TW,), jnp.int32),
        pltpu.VMEM((_TW, _D), jnp.float32),
        pltpu.VMEM((_TW, _D), jnp.float32),
        pltpu.VMEM((_TW, _D), jnp.float32),
        pltpu.VMEM((_L,), jnp.float32),
        pltpu.SemaphoreType.DMA,
    ],
)
def _margin_sc(emb, anc, pos, neg, out, *rest):
    _margin_body(emb, anc, pos, neg, out, *rest)


def kernel(embeddings, target, triplets):
    del target
    anc = triplets[:, 0]
    pos = triplets[:, 1]
    neg = triplets[:, 2]
    partials = _margin_sc(embeddings, anc, pos, neg)
    loss = partials[:, 0].sum() / partials[:, 1].sum()
    return (loss, triplets.shape[0])


# speedup vs baseline: 14.5778x; 1.0908x over previous
import functools

import jax
import jax.numpy as jnp
from jax import lax
from jax.experimental import pallas as pl
from jax.experimental.pallas import tpu as pltpu
from jax.experimental.pallas import tpu_sc as plsc

_MARGIN = 0.2
_BETA = 1.2

_NC = 2
_NS = 16
_NW = _NC * _NS
_L = 16

_N = 16384
_D = 16
_T = 65536
_TW = _T // _NW
_CH = 128
_NCHUNK = _TW // _CH
_K = 4


def _sqrt16(x):
    i = plsc.bitcast(x, jnp.int32)
    i = jnp.int32(0x5F3759DF) - (i >> 1)
    y = plsc.bitcast(i, jnp.float32)
    xh = x * 0.5
    y = y * (1.5 - xh * y * y)
    y = y * (1.5 - xh * y * y)
    y = y * (1.5 - xh * y * y)
    return x * y


def _margin_body(emb, anc, pos, neg, out, idx_a, idx_p, idx_n,
                 rows_a, rows_p, rows_n, obuf, *sems):
    wid = lax.axis_index("s") * _NC + lax.axis_index("c")
    base = wid * _TW

    pltpu.sync_copy(anc.at[pl.ds(base, _TW)], idx_a)
    pltpu.sync_copy(pos.at[pl.ds(base, _TW)], idx_p)
    pltpu.sync_copy(neg.at[pl.ds(base, _TW)], idx_n)

    def fire(c, sem_c):
        o = c * _CH
        pltpu.async_copy(emb.at[idx_a.at[pl.ds(o, _CH)]],
                         rows_a.at[pl.ds(o, _CH)], sem_c)
        pltpu.async_copy(emb.at[idx_p.at[pl.ds(o, _CH)]],
                         rows_p.at[pl.ds(o, _CH)], sem_c)
        pltpu.async_copy(emb.at[idx_n.at[pl.ds(o, _CH)]],
                         rows_n.at[pl.ds(o, _CH)], sem_c)

    def drain(c, sem_c):
        o = c * _CH
        pltpu.make_async_copy(emb.at[idx_a.at[pl.ds(o, _CH)]],
                              rows_a.at[pl.ds(o, _CH)], sem_c).wait()
        pltpu.make_async_copy(emb.at[idx_p.at[pl.ds(o, _CH)]],
                              rows_p.at[pl.ds(o, _CH)], sem_c).wait()
        pltpu.make_async_copy(emb.at[idx_n.at[pl.ds(o, _CH)]],
                              rows_n.at[pl.ds(o, _CH)], sem_c).wait()

    lanes = jnp.arange(_L, dtype=jnp.int32)
    zero = jnp.zeros((_L,), jnp.float32)

    def group(g, carry):
        asum, acnt = carry
        t0 = g * _L
        sap = zero
        san = zero
        for i in range(_L):
            va = rows_a[t0 + i, :]
            dap = va - rows_p[t0 + i, :]
            dan = va - rows_n[t0 + i, :]
            sap = jnp.where(lanes == i, jnp.sum(dap * dap), sap)
            san = jnp.where(lanes == i, jnp.sum(dan * dan), san)
        x_ap = sap + 1e-6
        x_an = san + 1e-6
        d_ap = _sqrt16(x_ap)
        d_an = _sqrt16(x_an)
        p_l = jnp.maximum(d_ap - (_BETA - _MARGIN), 0.0)
        n_l = jnp.maximum((_BETA + _MARGIN) - d_an, 0.0)
        hit = (p_l > 0.0) | (n_l > 0.0)
        asum = asum + (p_l + n_l)
        acnt = acnt + jnp.where(hit, 1.0, 0.0)
        return (asum, acnt)

    for k in range(_K):
        fire(k, sems[k])

    def outer(o, carry):
        for k in range(_K):
            c = o * _K + k
            drain(c, sems[k])

            @pl.when(o < _NCHUNK // _K - 1)
            def _():
                fire(c + _K, sems[k])

            def chunk_group(g, carry):
                return group(c * (_CH // _L) + g, carry)

            carry = lax.fori_loop(0, _CH // _L, chunk_group, carry)
        return carry

    asum, acnt = lax.fori_loop(0, _NCHUNK // _K, outer, (zero, zero))

    ssum = jnp.sum(asum)
    scnt = jnp.sum(acnt)
    obuf[...] = jnp.where(lanes == 0, ssum, jnp.where(lanes == 1, scnt, 0.0))
    pltpu.sync_copy(obuf, out.at[wid])


@functools.partial(
    pl.kernel,
    out_type=jax.ShapeDtypeStruct((_NW, _L), jnp.float32),
    mesh=plsc.VectorSubcoreMesh(core_axis_name="c", subcore_axis_name="s"),
    compiler_params=pltpu.CompilerParams(
        needs_layout_passes=False, use_tc_tiling_on_sc=False),
    scratch_types=[
        pltpu.VMEM((_TW,), jnp.int32),
        pltpu.VMEM((_TW,), jnp.int32),
        pltpu.VMEM((_TW,), jnp.int32),
        pltpu.VMEM((_TW, _D), jnp.float32),
        pltpu.VMEM((_TW, _D), jnp.float32),
        pltpu.VMEM((_TW, _D), jnp.float32),
        pltpu.VMEM((_L,), jnp.float32),
    ] + [pltpu.SemaphoreType.DMA] * _K,
)
def _margin_sc(emb, anc, pos, neg, out, *rest):
    _margin_body(emb, anc, pos, neg, out, *rest)


def kernel(embeddings, target, triplets):
    del target
    anc = triplets[:, 0]
    pos = triplets[:, 1]
    neg = triplets[:, 2]
    partials = _margin_sc(embeddings, anc, pos, neg)
    loss = partials[:, 0].sum() / partials[:, 1].sum()
    return (loss, triplets.shape[0])


# speedup vs baseline: 14.7621x; 1.0126x over previous
import functools

import jax
import jax.numpy as jnp
from jax import lax
from jax.experimental import pallas as pl
from jax.experimental.pallas import tpu as pltpu
from jax.experimental.pallas import tpu_sc as plsc

_MARGIN = 0.2
_BETA = 1.2

_NC = 2
_NS = 16
_NW = _NC * _NS
_L = 16

_N = 16384
_D = 16
_T = 65536
_TW = _T // _NW
_CH = 128
_NCHUNK = _TW // _CH
_K = 4


def _sqrt16(x):
    i = plsc.bitcast(x, jnp.int32)
    i = jnp.int32(0x5F3759DF) - (i >> 1)
    y = plsc.bitcast(i, jnp.float32)
    xh = x * 0.5
    y = y * (1.5 - xh * y * y)
    y = y * (1.5 - xh * y * y)
    y = y * (1.5 - xh * y * y)
    return x * y


def _margin_body(emb, anc, pos, neg, out, idx_a, idx_p, idx_n,
                 rows_a, rows_p, rows_n, obuf, *sems):
    wid = lax.axis_index("s") * _NC + lax.axis_index("c")
    base = wid * _TW

    pltpu.sync_copy(anc.at[pl.ds(base, _TW)], idx_a)
    pltpu.sync_copy(pos.at[pl.ds(base, _TW)], idx_p)
    pltpu.sync_copy(neg.at[pl.ds(base, _TW)], idx_n)

    def fire(c, sem_c):
        o = c * _CH
        pltpu.async_copy(emb.at[idx_a.at[pl.ds(o, _CH)]],
                         rows_a.at[pl.ds(o, _CH)], sem_c)
        pltpu.async_copy(emb.at[idx_p.at[pl.ds(o, _CH)]],
                         rows_p.at[pl.ds(o, _CH)], sem_c)
        pltpu.async_copy(emb.at[idx_n.at[pl.ds(o, _CH)]],
                         rows_n.at[pl.ds(o, _CH)], sem_c)

    def drain(c, sem_c):
        o = c * _CH
        pltpu.make_async_copy(emb.at[idx_a.at[pl.ds(o, _CH)]],
                              rows_a.at[pl.ds(o, _CH)], sem_c).wait()
        pltpu.make_async_copy(emb.at[idx_p.at[pl.ds(o, _CH)]],
                              rows_p.at[pl.ds(o, _CH)], sem_c).wait()
        pltpu.make_async_copy(emb.at[idx_n.at[pl.ds(o, _CH)]],
                              rows_n.at[pl.ds(o, _CH)], sem_c).wait()

    lanes = jnp.arange(_L, dtype=jnp.int32)
    zero = jnp.zeros((_L,), jnp.float32)

    def _perm(v, p):
        return jnp.take_along_axis(v, p, axis=0, mode="promise_in_bounds")

    def _tree_reduce(vs):
        d = _L // 2
        while len(vs) > 1:
            mask = (lanes & d) == 0
            perm = lanes ^ d
            vs = [jnp.where(mask, a, _perm(b, perm))
                  + jnp.where(mask, _perm(a, perm), b)
                  for a, b in zip(vs[0::2], vs[1::2])]
            d //= 2
        return vs[0]

    def group(g, carry):
        asum, acnt = carry
        t0 = g * _L
        qa = []
        qb = []
        for i in range(_L):
            va = rows_a[t0 + i, :]
            dap = va - rows_p[t0 + i, :]
            dan = va - rows_n[t0 + i, :]
            qa.append(dap * dap)
            qb.append(dan * dan)
        x_ap = _tree_reduce(qa) + 1e-6
        x_an = _tree_reduce(qb) + 1e-6
        d_ap = _sqrt16(x_ap)
        d_an = _sqrt16(x_an)
        p_l = jnp.maximum(d_ap - (_BETA - _MARGIN), 0.0)
        n_l = jnp.maximum((_BETA + _MARGIN) - d_an, 0.0)
        hit = (p_l > 0.0) | (n_l > 0.0)
        asum = asum + (p_l + n_l)
        acnt = acnt + jnp.where(hit, 1.0, 0.0)
        return (asum, acnt)

    for k in range(_K):
        fire(k, sems[k])

    def outer(o, carry):
        for k in range(_K):
            c = o * _K + k
            drain(c, sems[k])

            @pl.when(o < _NCHUNK // _K - 1)
            def _():
                fire(c + _K, sems[k])

            def chunk_group(g, carry):
                return group(c * (_CH // _L) + g, carry)

            carry = lax.fori_loop(0, _CH // _L, chunk_group, carry)
        return carry

    asum, acnt = lax.fori_loop(0, _NCHUNK // _K, outer, (zero, zero))

    ssum = jnp.sum(asum)
    scnt = jnp.sum(acnt)
    obuf[...] = jnp.where(lanes == 0, ssum, jnp.where(lanes == 1, scnt, 0.0))
    pltpu.sync_copy(obuf, out.at[wid])


@functools.partial(
    pl.kernel,
    out_type=jax.ShapeDtypeStruct((_NW, _L), jnp.float32),
    mesh=plsc.VectorSubcoreMesh(core_axis_name="c", subcore_axis_name="s"),
    compiler_params=pltpu.CompilerParams(
        needs_layout_passes=False, use_tc_tiling_on_sc=False),
    scratch_types=[
        pltpu.VMEM((_TW,), jnp.int32),
        pltpu.VMEM((_TW,), jnp.int32),
        pltpu.VMEM((_TW,), jnp.int32),
        pltpu.VMEM((_TW, _D), jnp.float32),
        pltpu.VMEM((_TW, _D), jnp.float32),
        pltpu.VMEM((_TW, _D), jnp.float32),
        pltpu.VMEM((_L,), jnp.float32),
    ] + [pltpu.SemaphoreType.DMA] * _K,
)
def _margin_sc(emb, anc, pos, neg, out, *rest):
    _margin_body(emb, anc, pos, neg, out, *rest)


def kernel(embeddings, target, triplets):
    del target
    anc = triplets[:, 0]
    pos = triplets[:, 1]
    neg = triplets[:, 2]
    partials = _margin_sc(embeddings, anc, pos, neg)
    loss = partials[:, 0].sum() / partials[:, 1].sum()
    return (loss, triplets.shape[0])
